# baseline (speedup 1.0000x reference)
import functools

import jax
import jax.numpy as jnp
from jax import lax
from jax.experimental import pallas as pl
from jax.experimental.pallas import tpu as pltpu
from jax.experimental.pallas import tpu_sc as plsc


def _proj_body(a_ref, wa_ref, wb_ref, pa_ref, pb_ref):
    a = a_ref[...]
    pa_ref[...] = jnp.dot(
        a, wa_ref[...], preferred_element_type=jnp.float32).astype(jnp.bfloat16)
    pb_ref[...] = jnp.dot(
        a, wb_ref[...], preferred_element_type=jnp.float32).astype(jnp.bfloat16)


def _node_proj(atom_fea, wa, wb):
    n = atom_fea.shape[0]
    d = wa.shape[1]
    return pl.pallas_call(
        _proj_body,
        out_shape=[jax.ShapeDtypeStruct((n, d), jnp.bfloat16)] * 2,
    )(atom_fea, wa, wb)


def _gather_sum_sc(pa, pb, nbr_t):
    e_total = nbr_t.shape[1]
    d = pa.shape[1]
    info = plsc.get_sparse_core_info()
    nc, ns = info.num_cores, info.num_subcores
    nw = nc * ns
    unit = 3200
    n_units = e_total // unit
    chunk = 800
    n_sub = unit // chunk
    max_u = (n_units + nw - 1) // nw
    assert n_units * unit == e_total
    mesh = plsc.VectorSubcoreMesh(core_axis_name="c", subcore_axis_name="s")

    @functools.partial(
        pl.kernel,
        mesh=mesh,
        out_type=jax.ShapeDtypeStruct((e_total // 4, 4 * d), jnp.bfloat16),
        scratch_types=[
            pltpu.VMEM((unit,), jnp.int32),
            pltpu.VMEM((unit,), jnp.int32),
            [pltpu.VMEM((chunk, d), jnp.bfloat16) for _ in range(2)],
            [pltpu.VMEM((chunk, d), jnp.bfloat16) for _ in range(2)],
            [pltpu.SemaphoreType.DMA for _ in range(2)],
            [pltpu.SemaphoreType.DMA for _ in range(2)],
            [pltpu.SemaphoreType.DMA for _ in range(2)],
        ],
        compiler_params=pltpu.CompilerParams(use_tc_tiling_on_sc=False),
    )
    def k(pa_hbm, pb_hbm, nbr_hbm, o_hbm, i0_v, i1_v, ga, gb, sa, sb, sw):
        wid = lax.axis_index("s") * nc + lax.axis_index("c")

        for u in range(max_u):
            uid = wid + nw * u

            @pl.when(uid < n_units)
            def _():
                b = uid // 4
                g = lax.rem(uid, 4)
                base_e = uid * unit

                pltpu.sync_copy(nbr_hbm.at[0, pl.ds(base_e, unit)], i0_v)
                pltpu.sync_copy(nbr_hbm.at[1, pl.ds(base_e, unit)], i1_v)

                def start_gather(sub):
                    st = sub % 2
                    ca = pltpu.async_copy(
                        pa_hbm.at[i0_v.at[pl.ds(sub * chunk, chunk)]],
                        ga[st], sa[st])
                    cb = pltpu.async_copy(
                        pb_hbm.at[i1_v.at[pl.ds(sub * chunk, chunk)]],
                        gb[st], sb[st])
                    return ca, cb

                gops = start_gather(0)
                wops = [None, None]
                for sub in range(n_sub):
                    st = sub % 2
                    nxt = None
                    if sub + 1 < n_sub:
                        if wops[(sub + 1) % 2] is not None:
                            wops[(sub + 1) % 2].wait()
                            wops[(sub + 1) % 2] = None
                        nxt = start_gather(sub + 1)
                    gops[0].wait()
                    gops[1].wait()
                    gops = nxt

                    ga_v, gb_v = ga[st], gb[st]

                    @pl.loop(0, chunk)
                    def _(r):
                        ga_v[r, :] = ga_v[r, :] + gb_v[r, :]

                    wops[st] = pltpu.async_copy(
                        ga_v,
                        o_hbm.at[pl.ds(b * unit + sub * chunk, chunk),
                                 pl.ds(g * d, d)],
                        sw[st])
                for w in wops:
                    if w is not None:
                        w.wait()

    return k(pa, pb, nbr_t)


def _edge_body(s_ref, et_ref, rt_ref, w12_ref, w3_ref, wr_ref, bias_ref,
               o_ref):
    s2t = s_ref[...].T
    b12 = bias_ref[0:32]
    b3 = bias_ref[32:48]
    br4 = bias_ref[48:64]
    bc = s_ref.shape[0]
    for g in range(4):
        stg = s2t[32 * g:32 * g + 32].astype(jnp.float32)
        etg = et_ref[:, g * bc:(g + 1) * bc]
        rtg = rt_ref[:, g * bc:(g + 1) * bc]
        z12 = stg + jnp.dot(w12_ref[...], etg,
                            preferred_element_type=jnp.float32) + b12
        z1 = z12[:16]
        z2 = z12[16:]
        h = (z1 * jax.nn.sigmoid(z1)) * jax.nn.sigmoid(z2)
        t = jnp.dot(w3_ref[...], h, preferred_element_type=jnp.float32) + b3
        gg = jnp.dot(wr_ref[...], rtg, preferred_element_type=jnp.float32) + br4
        o_ref[:, g * bc:(g + 1) * bc] = (t * jax.nn.sigmoid(t)) * gg


def _edge_mlp_t(s128, et, rt, w12, w3, wr, bias_col):
    e_total = et.shape[1]
    be = 12800
    bc = be // 4
    nb = e_total // be
    assert e_total % be == 0
    small = lambda shp: pl.BlockSpec(shp, lambda b: (0, 0))
    return pl.pallas_call(
        _edge_body,
        grid=(nb,),
        in_specs=[
            pl.BlockSpec((bc, 128), lambda b: (b, 0)),
            pl.BlockSpec((16, be), lambda b: (0, b)),
            pl.BlockSpec((16, be), lambda b: (0, b)),
            small((32, 16)),
            small((16, 16)),
            small((16, 16)),
            small((64, 1)),
        ],
        out_specs=pl.BlockSpec((16, be), lambda b: (0, b)),
        out_shape=jax.ShapeDtypeStruct((16, e_total), jnp.float32),
    )(s128, et, rt, w12, w3, wr, bias_col)


def kernel(atom_fea, edge_ij, nbr_atoms, bonds_r, W1, b1, W2, b2, Wr, br, W3, b3):
    f = atom_fea.shape[1]
    e_total = edge_ij.shape[0]
    eq = e_total // 4
    wa = jnp.concatenate([W1[:, :f].T, W2[:, :f].T], axis=1)
    wb = jnp.concatenate([W1[:, f:2 * f].T, W2[:, f:2 * f].T], axis=1)
    w12 = jnp.concatenate([W1[:, 2 * f:], W2[:, 2 * f:]], axis=0)
    bias_col = jnp.concatenate([b1, b2, b3, br])[:, None]
    pa, pb = _node_proj(atom_fea, wa, wb)
    s128 = _gather_sum_sc(pa, pb, nbr_atoms.T)
    ot = _edge_mlp_t(s128, edge_ij.T, bonds_r.T, w12, W3, Wr, bias_col)
    return ot.T


# speedup vs baseline: 1.5262x; 1.5262x over previous
import functools

import jax
import jax.numpy as jnp
from jax import lax
from jax.experimental import pallas as pl
from jax.experimental.pallas import tpu as pltpu
from jax.experimental.pallas import tpu_sc as plsc


def _proj_body(a_ref, wa_ref, wb_ref, pa_ref, pb_ref):
    a = a_ref[...]
    pa_ref[...] = jnp.dot(a, wa_ref[...], preferred_element_type=jnp.float32)
    pb_ref[...] = jnp.dot(a, wb_ref[...], preferred_element_type=jnp.float32)


def _node_proj(atom_fea, wa, wb):
    n = atom_fea.shape[0]
    d = wa.shape[1]
    return pl.pallas_call(
        _proj_body,
        out_shape=[jax.ShapeDtypeStruct((n, d), jnp.float32)] * 2,
    )(atom_fea, wa, wb)


def _gather_sum_sc(pa, pb, nbr_t, unit_lo, unit_hi):
    d = pa.shape[1]
    info = plsc.get_sparse_core_info()
    nc, ns = info.num_cores, info.num_subcores
    nw = nc * ns
    unit = 3200
    n_my = unit_hi - unit_lo
    chunk = 800
    n_sub = unit // chunk
    max_u = (n_my + nw - 1) // nw
    assert unit_lo % 4 == 0 and unit_hi % 4 == 0
    mesh = plsc.VectorSubcoreMesh(core_axis_name="c", subcore_axis_name="s")

    @functools.partial(
        pl.kernel,
        mesh=mesh,
        out_type=jax.ShapeDtypeStruct((n_my * unit // 4, 4 * d), jnp.float32),
        scratch_types=[
            pltpu.VMEM((unit,), jnp.int32),
            pltpu.VMEM((unit,), jnp.int32),
            [pltpu.VMEM((chunk, d), jnp.float32) for _ in range(2)],
            [pltpu.VMEM((chunk, d), jnp.float32) for _ in range(2)],
            [pltpu.SemaphoreType.DMA for _ in range(2)],
            [pltpu.SemaphoreType.DMA for _ in range(2)],
            [pltpu.SemaphoreType.DMA for _ in range(2)],
        ],
        compiler_params=pltpu.CompilerParams(use_tc_tiling_on_sc=False),
    )
    def k(pa_hbm, pb_hbm, nbr_hbm, o_hbm, i0_v, i1_v, ga, gb, sa, sb, sw):
        wid = lax.axis_index("s") * nc + lax.axis_index("c")

        for u in range(max_u):
            uid = unit_lo + wid + nw * u

            @pl.when(uid < unit_hi)
            def _():
                b = uid // 4 - unit_lo // 4
                g = lax.rem(uid, 4)
                base_e = uid * unit

                pltpu.sync_copy(nbr_hbm.at[0, pl.ds(base_e, unit)], i0_v)
                pltpu.sync_copy(nbr_hbm.at[1, pl.ds(base_e, unit)], i1_v)

                def start_gather(sub):
                    st = sub % 2
                    ca = pltpu.async_copy(
                        pa_hbm.at[i0_v.at[pl.ds(sub * chunk, chunk)]],
                        ga[st], sa[st])
                    cb = pltpu.async_copy(
                        pb_hbm.at[i1_v.at[pl.ds(sub * chunk, chunk)]],
                        gb[st], sb[st])
                    return ca, cb

                gops = start_gather(0)
                wops = [None, None]
                for sub in range(n_sub):
                    st = sub % 2
                    nxt = None
                    if sub + 1 < n_sub:
                        if wops[(sub + 1) % 2] is not None:
                            wops[(sub + 1) % 2].wait()
                            wops[(sub + 1) % 2] = None
                        nxt = start_gather(sub + 1)
                    gops[0].wait()
                    gops[1].wait()
                    gops = nxt

                    ga_v, gb_v = ga[st], gb[st]

                    @pl.loop(0, chunk)
                    def _(r):
                        for c in range(0, d, 16):
                            slc = (pl.ds(r, 1), pl.ds(c, 16))
                            ga_v.at[*slc][...] = (ga_v.at[*slc][...]
                                                  + gb_v.at[*slc][...])

                    wops[st] = pltpu.async_copy(
                        ga_v,
                        o_hbm.at[pl.ds(b * unit + sub * chunk, chunk),
                                 pl.ds(g * d, d)],
                        sw[st])
                for w in wops:
                    if w is not None:
                        w.wait()

    return k(pa, pb, nbr_t)


def _edge_body(s_ref, et_ref, rt_ref, w12_ref, w3_ref, wr_ref, bias_ref,
               o_ref):
    s2t = s_ref[...].T
    b12 = bias_ref[0:32]
    b3 = bias_ref[32:48]
    br4 = bias_ref[48:64]
    bc = s_ref.shape[0]
    for g in range(4):
        stg = s2t[32 * g:32 * g + 32]
        etg = et_ref[:, g * bc:(g + 1) * bc]
        rtg = rt_ref[:, g * bc:(g + 1) * bc]
        z12 = stg + jnp.dot(w12_ref[...], etg,
                            preferred_element_type=jnp.float32) + b12
        z1 = z12[:16]
        z2 = z12[16:]
        h = (z1 * jax.nn.sigmoid(z1)) * jax.nn.sigmoid(z2)
        t = jnp.dot(w3_ref[...], h, preferred_element_type=jnp.float32) + b3
        gg = jnp.dot(wr_ref[...], rtg, preferred_element_type=jnp.float32) + br4
        o_ref[:, g * bc:(g + 1) * bc] = (t * jax.nn.sigmoid(t)) * gg


def _edge_mlp_t(s128, et, rt, w12, w3, wr, bias_col, blk_lo, n_blk,
                o_prev=None):
    e_total = et.shape[1]
    be = 12800
    bc = be // 4
    small = lambda shp: pl.BlockSpec(shp, lambda b: (0, 0))
    in_specs = [
        pl.BlockSpec((bc, 128), lambda b: (b, 0)),
        pl.BlockSpec((16, be), lambda b: (0, blk_lo + b)),
        pl.BlockSpec((16, be), lambda b: (0, blk_lo + b)),
        small((32, 16)),
        small((16, 16)),
        small((16, 16)),
        small((64, 1)),
    ]
    args = [s128, et, rt, w12, w3, wr, bias_col]
    kwargs = {}
    body = _edge_body
    if o_prev is not None:
        in_specs.append(pl.BlockSpec(memory_space=pl.ANY))
        args.append(o_prev)
        kwargs["input_output_aliases"] = {7: 0}
        body = lambda *refs: _edge_body(*refs[:7], refs[-1])
    return pl.pallas_call(
        body,
        grid=(n_blk,),
        in_specs=in_specs,
        out_specs=pl.BlockSpec((16, be), lambda b: (0, blk_lo + b)),
        out_shape=jax.ShapeDtypeStruct((16, e_total), jnp.float32),
        **kwargs,
    )(*args)


def kernel(atom_fea, edge_ij, nbr_atoms, bonds_r, W1, b1, W2, b2, Wr, br, W3, b3):
    f = atom_fea.shape[1]
    e_total = edge_ij.shape[0]
    eq = e_total // 4
    wa = jnp.concatenate([W1[:, :f].T, W2[:, :f].T], axis=1)
    wb = jnp.concatenate([W1[:, f:2 * f].T, W2[:, f:2 * f].T], axis=1)
    w12 = jnp.concatenate([W1[:, 2 * f:], W2[:, 2 * f:]], axis=0)
    bias_col = jnp.concatenate([b1, b2, b3, br])[:, None]
    pa, pb = _node_proj(atom_fea, wa, wb)
    nbr_t = nbr_atoms.T
    et = edge_ij.T
    rt = bonds_r.T
    chunks = [(0, 32, 0, 8), (32, 64, 8, 8), (64, 100, 16, 9)]
    ss = [_gather_sum_sc(pa, pb, nbr_t, lo, hi) for lo, hi, _, _ in chunks]
    ot = None
    for s128, (_, _, blk_lo, n_blk) in zip(ss, chunks):
        ot = _edge_mlp_t(s128, et, rt, w12, W3, Wr, bias_col,
                         blk_lo, n_blk, o_prev=ot)
    return ot.T


# speedup vs baseline: 1.7888x; 1.1720x over previous
import functools

import jax
import jax.numpy as jnp
from jax import lax
from jax.experimental import pallas as pl
from jax.experimental.pallas import tpu as pltpu
from jax.experimental.pallas import tpu_sc as plsc


def _proj_body(a_ref, wa_ref, wb_ref, pa_ref, pb_ref):
    a = a_ref[...]
    pa_ref[...] = jnp.dot(a, wa_ref[...], preferred_element_type=jnp.float32)
    pb_ref[...] = jnp.dot(a, wb_ref[...], preferred_element_type=jnp.float32)


def _node_proj(atom_fea, wa, wb):
    n = atom_fea.shape[0]
    d = wa.shape[1]
    return pl.pallas_call(
        _proj_body,
        out_shape=[jax.ShapeDtypeStruct((n, d), jnp.float32)] * 2,
    )(atom_fea, wa, wb)


def _gather_sum_sc(pa, pb, nbr_t, unit_lo, unit_hi):
    d = pa.shape[1]
    info = plsc.get_sparse_core_info()
    nc, ns = info.num_cores, info.num_subcores
    nw = nc * ns
    unit = 3200
    n_my = unit_hi - unit_lo
    chunk = 400
    spu = unit // chunk
    npw = n_my * spu // nw
    assert unit_lo % 4 == 0 and unit_hi % 4 == 0 and npw * nw == n_my * spu
    mesh = plsc.VectorSubcoreMesh(core_axis_name="c", subcore_axis_name="s")

    @functools.partial(
        pl.kernel,
        mesh=mesh,
        out_type=jax.ShapeDtypeStruct((n_my * unit // 4, 4 * d), jnp.float32),
        scratch_types=[
            pltpu.VMEM((npw * chunk,), jnp.int32),
            pltpu.VMEM((npw * chunk,), jnp.int32),
            [pltpu.VMEM((chunk, d), jnp.float32) for _ in range(2)],
            [pltpu.VMEM((chunk, d), jnp.float32) for _ in range(2)],
            [pltpu.SemaphoreType.DMA for _ in range(2)],
            [pltpu.SemaphoreType.DMA for _ in range(2)],
            [pltpu.SemaphoreType.DMA for _ in range(2)],
        ],
        compiler_params=pltpu.CompilerParams(use_tc_tiling_on_sc=False),
    )
    def k(pa_hbm, pb_hbm, nbr_hbm, o_hbm, i0_v, i1_v, ga, gb, sa, sb, sw):
        wid = lax.axis_index("s") * nc + lax.axis_index("c")
        sid0 = wid * npw
        e0 = unit_lo * unit + sid0 * chunk
        pltpu.sync_copy(nbr_hbm.at[0, pl.ds(e0, npw * chunk)], i0_v)
        pltpu.sync_copy(nbr_hbm.at[1, pl.ds(e0, npw * chunk)], i1_v)

        def start_gather(kk):
            st = kk % 2
            ca = pltpu.async_copy(
                pa_hbm.at[i0_v.at[pl.ds(kk * chunk, chunk)]], ga[st], sa[st])
            cb = pltpu.async_copy(
                pb_hbm.at[i1_v.at[pl.ds(kk * chunk, chunk)]], gb[st], sb[st])
            return ca, cb

        gops = start_gather(0)
        wops = [None, None]
        for kk in range(npw):
            st = kk % 2
            nxt = None
            if kk + 1 < npw:
                if wops[(kk + 1) % 2] is not None:
                    wops[(kk + 1) % 2].wait()
                    wops[(kk + 1) % 2] = None
                nxt = start_gather(kk + 1)
            gops[0].wait()
            gops[1].wait()
            gops = nxt

            ga_v, gb_v = ga[st], gb[st]

            @pl.loop(0, chunk)
            def _(r):
                for c in range(0, d, 16):
                    slc = (pl.ds(r, 1), pl.ds(c, 16))
                    ga_v.at[*slc][...] = (ga_v.at[*slc][...]
                                          + gb_v.at[*slc][...])

            sid = sid0 + kk
            uid_rel = sid // spu
            g = lax.rem(unit_lo + uid_rel, 4)
            row0 = (uid_rel // 4) * unit + lax.rem(sid, spu) * chunk
            wops[st] = pltpu.async_copy(
                ga_v,
                o_hbm.at[pl.ds(row0, chunk), pl.ds(g * d, d)],
                sw[st])
        for w in wops:
            if w is not None:
                w.wait()

    return k(pa, pb, nbr_t)


def _edge_body(s_ref, et_ref, rt_ref, w12_ref, w3_ref, wr_ref, bias_ref,
               o_ref):
    s2t = s_ref[...].T
    b12 = bias_ref[0:32]
    b3 = bias_ref[32:48]
    br4 = bias_ref[48:64]
    bc = s_ref.shape[0]
    for g in range(4):
        stg = s2t[32 * g:32 * g + 32]
        etg = et_ref[:, g * bc:(g + 1) * bc]
        rtg = rt_ref[:, g * bc:(g + 1) * bc]
        z12 = stg + jnp.dot(w12_ref[...], etg,
                            preferred_element_type=jnp.float32) + b12
        z1 = z12[:16]
        z2 = z12[16:]
        h = (z1 * jax.nn.sigmoid(z1)) * jax.nn.sigmoid(z2)
        t = jnp.dot(w3_ref[...], h, preferred_element_type=jnp.float32) + b3
        gg = jnp.dot(wr_ref[...], rtg, preferred_element_type=jnp.float32) + br4
        o_ref[:, g * bc:(g + 1) * bc] = (t * jax.nn.sigmoid(t)) * gg


def _edge_mlp_t(s128, et, rt, w12, w3, wr, bias_col, blk_lo, n_blk,
                o_prev=None):
    e_total = et.shape[1]
    be = 12800
    bc = be // 4
    small = lambda shp: pl.BlockSpec(shp, lambda b: (0, 0))
    in_specs = [
        pl.BlockSpec((bc, 128), lambda b: (b, 0)),
        pl.BlockSpec((16, be), lambda b: (0, blk_lo + b)),
        pl.BlockSpec((16, be), lambda b: (0, blk_lo + b)),
        small((32, 16)),
        small((16, 16)),
        small((16, 16)),
        small((64, 1)),
    ]
    args = [s128, et, rt, w12, w3, wr, bias_col]
    kwargs = {}
    body = _edge_body
    if o_prev is not None:
        in_specs.append(pl.BlockSpec(memory_space=pl.ANY))
        args.append(o_prev)
        kwargs["input_output_aliases"] = {7: 0}
        body = lambda *refs: _edge_body(*refs[:7], refs[-1])
    return pl.pallas_call(
        body,
        grid=(n_blk,),
        in_specs=in_specs,
        out_specs=pl.BlockSpec((16, be), lambda b: (0, blk_lo + b)),
        out_shape=jax.ShapeDtypeStruct((16, e_total), jnp.float32),
        **kwargs,
    )(*args)


def kernel(atom_fea, edge_ij, nbr_atoms, bonds_r, W1, b1, W2, b2, Wr, br, W3, b3):
    f = atom_fea.shape[1]
    e_total = edge_ij.shape[0]
    eq = e_total // 4
    wa = jnp.concatenate([W1[:, :f].T, W2[:, :f].T], axis=1)
    wb = jnp.concatenate([W1[:, f:2 * f].T, W2[:, f:2 * f].T], axis=1)
    w12 = jnp.concatenate([W1[:, 2 * f:], W2[:, 2 * f:]], axis=0)
    bias_col = jnp.concatenate([b1, b2, b3, br])[:, None]
    pa, pb = _node_proj(atom_fea, wa, wb)
    nbr_t = nbr_atoms.T
    et = edge_ij.T
    rt = bonds_r.T
    chunks = [(0, 32, 0, 8), (32, 64, 8, 8), (64, 100, 16, 9)]
    ss = [_gather_sum_sc(pa, pb, nbr_t, lo, hi) for lo, hi, _, _ in chunks]
    ot = None
    for s128, (_, _, blk_lo, n_blk) in zip(ss, chunks):
        ot = _edge_mlp_t(s128, et, rt, w12, W3, Wr, bias_col,
                         blk_lo, n_blk, o_prev=ot)
    return ot.T


# speedup vs baseline: 1.8083x; 1.0109x over previous
import functools

import jax
import jax.numpy as jnp
from jax import lax
from jax.experimental import pallas as pl
from jax.experimental.pallas import tpu as pltpu
from jax.experimental.pallas import tpu_sc as plsc


def _proj_body(a_ref, wa_ref, wb_ref, pa_ref, pb_ref):
    a = a_ref[...]
    pa_ref[...] = jnp.dot(a, wa_ref[...], preferred_element_type=jnp.float32)
    pb_ref[...] = jnp.dot(a, wb_ref[...], preferred_element_type=jnp.float32)


def _node_proj(atom_fea, wa, wb):
    n = atom_fea.shape[0]
    d = wa.shape[1]
    return pl.pallas_call(
        _proj_body,
        out_shape=[jax.ShapeDtypeStruct((n, d), jnp.float32)] * 2,
    )(atom_fea, wa, wb)


def _gather_sum_sc(pa, pb, nbr_t, unit_lo, unit_hi):
    d = pa.shape[1]
    info = plsc.get_sparse_core_info()
    nc, ns = info.num_cores, info.num_subcores
    nw = nc * ns
    unit = 3200
    n_my = unit_hi - unit_lo
    chunk = 400
    spu = unit // chunk
    npw = n_my * spu // nw
    assert unit_lo % 4 == 0 and unit_hi % 4 == 0 and npw * nw == n_my * spu
    mesh = plsc.VectorSubcoreMesh(core_axis_name="c", subcore_axis_name="s")

    @functools.partial(
        pl.kernel,
        mesh=mesh,
        out_type=jax.ShapeDtypeStruct((n_my * unit // 4, 4 * d), jnp.float32),
        scratch_types=[
            pltpu.VMEM((npw * chunk,), jnp.int32),
            pltpu.VMEM((npw * chunk,), jnp.int32),
            [pltpu.VMEM((chunk, d), jnp.float32) for _ in range(2)],
            [pltpu.VMEM((chunk, d), jnp.float32) for _ in range(2)],
            [pltpu.SemaphoreType.DMA for _ in range(2)],
            [pltpu.SemaphoreType.DMA for _ in range(2)],
            [pltpu.SemaphoreType.DMA for _ in range(2)],
        ],
        compiler_params=pltpu.CompilerParams(use_tc_tiling_on_sc=False),
    )
    def k(pa_hbm, pb_hbm, nbr_hbm, o_hbm, i0_v, i1_v, ga, gb, sa, sb, sw):
        wid = lax.axis_index("s") * nc + lax.axis_index("c")
        sid0 = wid * npw
        e0 = unit_lo * unit + sid0 * chunk
        pltpu.sync_copy(nbr_hbm.at[0, pl.ds(e0, npw * chunk)], i0_v)
        pltpu.sync_copy(nbr_hbm.at[1, pl.ds(e0, npw * chunk)], i1_v)

        def start_gather(kk):
            st = kk % 2
            ca = pltpu.async_copy(
                pa_hbm.at[i0_v.at[pl.ds(kk * chunk, chunk)]], ga[st], sa[st])
            cb = pltpu.async_copy(
                pb_hbm.at[i1_v.at[pl.ds(kk * chunk, chunk)]], gb[st], sb[st])
            return ca, cb

        gops = start_gather(0)
        wops = [None, None]
        for kk in range(npw):
            st = kk % 2
            nxt = None
            if kk + 1 < npw:
                if wops[(kk + 1) % 2] is not None:
                    wops[(kk + 1) % 2].wait()
                    wops[(kk + 1) % 2] = None
                nxt = start_gather(kk + 1)
            gops[0].wait()
            gops[1].wait()
            gops = nxt

            ga_v, gb_v = ga[st], gb[st]

            @pl.loop(0, chunk)
            def _(r):
                for c in range(0, d, 16):
                    slc = (pl.ds(r, 1), pl.ds(c, 16))
                    ga_v.at[*slc][...] = (ga_v.at[*slc][...]
                                          + gb_v.at[*slc][...])

            sid = sid0 + kk
            uid_rel = sid // spu
            g = lax.rem(unit_lo + uid_rel, 4)
            row0 = (uid_rel // 4) * unit + lax.rem(sid, spu) * chunk
            wops[st] = pltpu.async_copy(
                ga_v,
                o_hbm.at[pl.ds(row0, chunk), pl.ds(g * d, d)],
                sw[st])
        for w in wops:
            if w is not None:
                w.wait()

    return k(pa, pb, nbr_t)


def _edge_body(s_ref, et_ref, rt_ref, w12_ref, w3_ref, wr_ref, bias_ref,
               o_ref):
    s2t = s_ref[...].T
    b12 = bias_ref[0:32]
    b3 = bias_ref[32:48]
    br4 = bias_ref[48:64]
    bc = s_ref.shape[0]
    for g in range(4):
        stg = s2t[32 * g:32 * g + 32]
        etg = et_ref[:, g * bc:(g + 1) * bc]
        rtg = rt_ref[:, g * bc:(g + 1) * bc]
        z12 = stg + jnp.dot(w12_ref[...], etg,
                            preferred_element_type=jnp.float32) + b12
        z1 = z12[:16]
        z2 = z12[16:]
        h = (z1 * jax.nn.sigmoid(z1)) * jax.nn.sigmoid(z2)
        t = jnp.dot(w3_ref[...], h, preferred_element_type=jnp.float32) + b3
        gg = jnp.dot(wr_ref[...], rtg, preferred_element_type=jnp.float32) + br4
        o_ref[:, g * bc:(g + 1) * bc] = (t * jax.nn.sigmoid(t)) * gg


def _edge_mlp_t(s128, et, rt, w12, w3, wr, bias_col, blk_lo, n_blk,
                o_prev=None):
    e_total = et.shape[1]
    be = 12800
    bc = be // 4
    small = lambda shp: pl.BlockSpec(shp, lambda b: (0, 0))
    in_specs = [
        pl.BlockSpec((bc, 128), lambda b: (b, 0)),
        pl.BlockSpec((16, be), lambda b: (0, blk_lo + b)),
        pl.BlockSpec((16, be), lambda b: (0, blk_lo + b)),
        small((32, 16)),
        small((16, 16)),
        small((16, 16)),
        small((64, 1)),
    ]
    args = [s128, et, rt, w12, w3, wr, bias_col]
    kwargs = {}
    body = _edge_body
    if o_prev is not None:
        in_specs.append(pl.BlockSpec(memory_space=pl.ANY))
        args.append(o_prev)
        kwargs["input_output_aliases"] = {7: 0}
        body = lambda *refs: _edge_body(*refs[:7], refs[-1])
    return pl.pallas_call(
        body,
        grid=(n_blk,),
        in_specs=in_specs,
        out_specs=pl.BlockSpec((16, be), lambda b: (0, blk_lo + b)),
        out_shape=jax.ShapeDtypeStruct((16, e_total), jnp.float32),
        **kwargs,
    )(*args)


def kernel(atom_fea, edge_ij, nbr_atoms, bonds_r, W1, b1, W2, b2, Wr, br, W3, b3):
    f = atom_fea.shape[1]
    e_total = edge_ij.shape[0]
    eq = e_total // 4
    wa = jnp.concatenate([W1[:, :f].T, W2[:, :f].T], axis=1)
    wb = jnp.concatenate([W1[:, f:2 * f].T, W2[:, f:2 * f].T], axis=1)
    w12 = jnp.concatenate([W1[:, 2 * f:], W2[:, 2 * f:]], axis=0)
    bias_col = jnp.concatenate([b1, b2, b3, br])[:, None]
    pa, pb = _node_proj(atom_fea, wa, wb)
    nbr_t = nbr_atoms.T
    et = edge_ij.T
    rt = bonds_r.T
    chunks = [(0, 32, 0, 8), (32, 64, 8, 8), (64, 88, 16, 6), (88, 100, 22, 3)]
    ss = [_gather_sum_sc(pa, pb, nbr_t, lo, hi) for lo, hi, _, _ in chunks]
    ot = None
    for s128, (_, _, blk_lo, n_blk) in zip(ss, chunks):
        ot = _edge_mlp_t(s128, et, rt, w12, W3, Wr, bias_col,
                         blk_lo, n_blk, o_prev=ot)
    return ot.T


# speedup vs baseline: 1.8400x; 1.0175x over previous
import functools

import jax
import jax.numpy as jnp
from jax import lax
from jax.experimental import pallas as pl
from jax.experimental.pallas import tpu as pltpu
from jax.experimental.pallas import tpu_sc as plsc


def _proj_body(a_ref, w1_ref, w2_ref, pa_ref, pb_ref):
    a = a_ref[...]
    f = a.shape[1]
    dn = (((1,), (1,)), ((), ()))

    def proj(w_sl):
        return lax.dot_general(a, w_sl, dn, preferred_element_type=jnp.float32)

    w1 = w1_ref[...]
    w2 = w2_ref[...]
    pa_ref[...] = jnp.concatenate(
        [proj(w1[:, :f]), proj(w2[:, :f])], axis=1)
    pb_ref[...] = jnp.concatenate(
        [proj(w1[:, f:2 * f]), proj(w2[:, f:2 * f])], axis=1)


def _node_proj(atom_fea, w1, w2):
    n = atom_fea.shape[0]
    return pl.pallas_call(
        _proj_body,
        out_shape=[jax.ShapeDtypeStruct((n, 32), jnp.float32)] * 2,
    )(atom_fea, w1, w2)


def _gather_sum_sc(pa, pb, nbr_t, unit_lo, unit_hi):
    d = pa.shape[1]
    info = plsc.get_sparse_core_info()
    nc, ns = info.num_cores, info.num_subcores
    nw = nc * ns
    unit = 3200
    n_my = unit_hi - unit_lo
    chunk = 400
    spu = unit // chunk
    npw = n_my * spu // nw
    assert unit_lo % 4 == 0 and unit_hi % 4 == 0 and npw * nw == n_my * spu
    mesh = plsc.VectorSubcoreMesh(core_axis_name="c", subcore_axis_name="s")

    @functools.partial(
        pl.kernel,
        mesh=mesh,
        out_type=jax.ShapeDtypeStruct((n_my * unit // 4, 4 * d), jnp.float32),
        scratch_types=[
            pltpu.VMEM((npw * chunk,), jnp.int32),
            pltpu.VMEM((npw * chunk,), jnp.int32),
            [pltpu.VMEM((chunk, d), jnp.float32) for _ in range(2)],
            [pltpu.VMEM((chunk, d), jnp.float32) for _ in range(2)],
            [pltpu.SemaphoreType.DMA for _ in range(2)],
            [pltpu.SemaphoreType.DMA for _ in range(2)],
            [pltpu.SemaphoreType.DMA for _ in range(2)],
        ],
        compiler_params=pltpu.CompilerParams(use_tc_tiling_on_sc=False),
    )
    def k(pa_hbm, pb_hbm, nbr_hbm, o_hbm, i0_v, i1_v, ga, gb, sa, sb, sw):
        wid = lax.axis_index("s") * nc + lax.axis_index("c")
        sid0 = wid * npw
        e0 = unit_lo * unit + sid0 * chunk
        pltpu.sync_copy(nbr_hbm.at[0, pl.ds(e0, npw * chunk)], i0_v)
        pltpu.sync_copy(nbr_hbm.at[1, pl.ds(e0, npw * chunk)], i1_v)

        def start_gather(kk):
            st = kk % 2
            ca = pltpu.async_copy(
                pa_hbm.at[i0_v.at[pl.ds(kk * chunk, chunk)]], ga[st], sa[st])
            cb = pltpu.async_copy(
                pb_hbm.at[i1_v.at[pl.ds(kk * chunk, chunk)]], gb[st], sb[st])
            return ca, cb

        gops = start_gather(0)
        wops = [None, None]
        for kk in range(npw):
            st = kk % 2
            nxt = None
            if kk + 1 < npw:
                if wops[(kk + 1) % 2] is not None:
                    wops[(kk + 1) % 2].wait()
                    wops[(kk + 1) % 2] = None
                nxt = start_gather(kk + 1)
            gops[0].wait()
            gops[1].wait()
            gops = nxt

            ga_v, gb_v = ga[st], gb[st]

            @pl.loop(0, chunk)
            def _(r):
                for c in range(0, d, 16):
                    slc = (pl.ds(r, 1), pl.ds(c, 16))
                    ga_v.at[*slc][...] = (ga_v.at[*slc][...]
                                          + gb_v.at[*slc][...])

            sid = sid0 + kk
            uid_rel = sid // spu
            g = lax.rem(unit_lo + uid_rel, 4)
            row0 = (uid_rel // 4) * unit + lax.rem(sid, spu) * chunk
            wops[st] = pltpu.async_copy(
                ga_v,
                o_hbm.at[pl.ds(row0, chunk), pl.ds(g * d, d)],
                sw[st])
        for w in wops:
            if w is not None:
                w.wait()

    return k(pa, pb, nbr_t)


def _edge_body(s_ref, et_ref, rt_ref, w12_ref, w3_ref, wr_ref, bias_ref,
               o_ref):
    s2t = s_ref[...].T
    b12 = bias_ref[0:32]
    b3 = bias_ref[32:48]
    br4 = bias_ref[48:64]
    bc = s_ref.shape[0]
    for g in range(4):
        stg = s2t[32 * g:32 * g + 32]
        etg = et_ref[:, g * bc:(g + 1) * bc]
        rtg = rt_ref[:, g * bc:(g + 1) * bc]
        z12 = stg + jnp.dot(w12_ref[...], etg,
                            preferred_element_type=jnp.float32) + b12
        z1 = z12[:16]
        z2 = z12[16:]
        h = (z1 * jax.nn.sigmoid(z1)) * jax.nn.sigmoid(z2)
        t = jnp.dot(w3_ref[...], h, preferred_element_type=jnp.float32) + b3
        gg = jnp.dot(wr_ref[...], rtg, preferred_element_type=jnp.float32) + br4
        o_ref[:, g * bc:(g + 1) * bc] = (t * jax.nn.sigmoid(t)) * gg


def _edge_mlp_t(s128, et, rt, w12, w3, wr, bias_col, blk_lo, n_blk,
                o_prev=None):
    e_total = et.shape[1]
    be = 12800
    bc = be // 4
    small = lambda shp: pl.BlockSpec(shp, lambda b: (0, 0))
    in_specs = [
        pl.BlockSpec((bc, 128), lambda b: (b, 0)),
        pl.BlockSpec((16, be), lambda b: (0, blk_lo + b)),
        pl.BlockSpec((16, be), lambda b: (0, blk_lo + b)),
        small((32, 16)),
        small((16, 16)),
        small((16, 16)),
        small((64, 1)),
    ]
    args = [s128, et, rt, w12, w3, wr, bias_col]
    kwargs = {}
    body = _edge_body
    if o_prev is not None:
        in_specs.append(pl.BlockSpec(memory_space=pl.ANY))
        args.append(o_prev)
        kwargs["input_output_aliases"] = {7: 0}
        body = lambda *refs: _edge_body(*refs[:7], refs[-1])
    return pl.pallas_call(
        body,
        grid=(n_blk,),
        in_specs=in_specs,
        out_specs=pl.BlockSpec((16, be), lambda b: (0, blk_lo + b)),
        out_shape=jax.ShapeDtypeStruct((16, e_total), jnp.float32),
        **kwargs,
    )(*args)


def kernel(atom_fea, edge_ij, nbr_atoms, bonds_r, W1, b1, W2, b2, Wr, br, W3, b3):
    f = atom_fea.shape[1]
    e_total = edge_ij.shape[0]
    eq = e_total // 4
    w12 = jnp.concatenate([W1[:, 2 * f:], W2[:, 2 * f:]], axis=0)
    bias_col = jnp.concatenate([b1, b2, b3, br])[:, None]
    pa, pb = _node_proj(atom_fea, W1, W2)
    nbr_t = nbr_atoms.T
    et = edge_ij.T
    rt = bonds_r.T
    chunks = [(0, 32, 0, 8), (32, 64, 8, 8), (64, 88, 16, 6), (88, 100, 22, 3)]
    ss = [_gather_sum_sc(pa, pb, nbr_t, lo, hi) for lo, hi, _, _ in chunks]
    ot = None
    for s128, (_, _, blk_lo, n_blk) in zip(ss, chunks):
        ot = _edge_mlp_t(s128, et, rt, w12, W3, Wr, bias_col,
                         blk_lo, n_blk, o_prev=ot)
    return ot.T


# speedup vs baseline: 1.8478x; 1.0042x over previous
import functools

import jax
import jax.numpy as jnp
from jax import lax
from jax.experimental import pallas as pl
from jax.experimental.pallas import tpu as pltpu
from jax.experimental.pallas import tpu_sc as plsc


def _proj_body(a_ref, w1_ref, w2_ref, pa_ref, pb_ref):
    a = a_ref[...]
    f = a.shape[1]
    dn = (((1,), (1,)), ((), ()))

    def proj(w_sl):
        return lax.dot_general(a, w_sl, dn, preferred_element_type=jnp.float32)

    w1 = w1_ref[...]
    w2 = w2_ref[...]
    pa_ref[...] = jnp.concatenate(
        [proj(w1[:, :f]), proj(w2[:, :f])], axis=1)
    pb_ref[...] = jnp.concatenate(
        [proj(w1[:, f:2 * f]), proj(w2[:, f:2 * f])], axis=1)


def _node_proj(atom_fea, w1, w2):
    n = atom_fea.shape[0]
    return pl.pallas_call(
        _proj_body,
        out_shape=[jax.ShapeDtypeStruct((n, 32), jnp.float32)] * 2,
    )(atom_fea, w1, w2)


def _gather_sum_sc(pa, pb, nbr_t, unit_lo, unit_hi):
    d = pa.shape[1]
    info = plsc.get_sparse_core_info()
    nc, ns = info.num_cores, info.num_subcores
    nw = nc * ns
    unit = 3200
    n_my = unit_hi - unit_lo
    chunk = 400
    spu = unit // chunk
    npw = n_my * spu // nw
    assert unit_lo % 4 == 0 and unit_hi % 4 == 0 and npw * nw == n_my * spu
    mesh = plsc.VectorSubcoreMesh(core_axis_name="c", subcore_axis_name="s")

    @functools.partial(
        pl.kernel,
        mesh=mesh,
        out_type=jax.ShapeDtypeStruct((n_my * unit // 4, 4 * d), jnp.float32),
        scratch_types=[
            pltpu.VMEM((npw * chunk,), jnp.int32),
            pltpu.VMEM((npw * chunk,), jnp.int32),
            [pltpu.VMEM((chunk, d), jnp.float32) for _ in range(2)],
            [pltpu.VMEM((chunk, d), jnp.float32) for _ in range(2)],
            [pltpu.SemaphoreType.DMA for _ in range(2)],
            [pltpu.SemaphoreType.DMA for _ in range(2)],
            [pltpu.SemaphoreType.DMA for _ in range(2)],
        ],
        compiler_params=pltpu.CompilerParams(use_tc_tiling_on_sc=False),
    )
    def k(pa_hbm, pb_hbm, nbr_hbm, o_hbm, i0_v, i1_v, ga, gb, sa, sb, sw):
        wid = lax.axis_index("s") * nc + lax.axis_index("c")
        sid0 = wid * npw
        e0 = unit_lo * unit + sid0 * chunk
        pltpu.sync_copy(nbr_hbm.at[0, pl.ds(e0, npw * chunk)], i0_v)
        pltpu.sync_copy(nbr_hbm.at[1, pl.ds(e0, npw * chunk)], i1_v)

        def start_gather(kk):
            st = kk % 2
            ca = pltpu.async_copy(
                pa_hbm.at[i0_v.at[pl.ds(kk * chunk, chunk)]], ga[st], sa[st])
            cb = pltpu.async_copy(
                pb_hbm.at[i1_v.at[pl.ds(kk * chunk, chunk)]], gb[st], sb[st])
            return ca, cb

        gops = start_gather(0)
        wops = [None, None]
        for kk in range(npw):
            st = kk % 2
            nxt = None
            if kk + 1 < npw:
                if wops[(kk + 1) % 2] is not None:
                    wops[(kk + 1) % 2].wait()
                    wops[(kk + 1) % 2] = None
                nxt = start_gather(kk + 1)
            gops[0].wait()
            gops[1].wait()
            gops = nxt

            ga_v, gb_v = ga[st], gb[st]

            @pl.loop(0, chunk)
            def _(r):
                for c in range(0, d, 16):
                    slc = (pl.ds(r, 1), pl.ds(c, 16))
                    ga_v.at[*slc][...] = (ga_v.at[*slc][...]
                                          + gb_v.at[*slc][...])

            sid = sid0 + kk
            uid_rel = sid // spu
            g = lax.rem(unit_lo + uid_rel, 4)
            row0 = (uid_rel // 4) * unit + lax.rem(sid, spu) * chunk
            wops[st] = pltpu.async_copy(
                ga_v,
                o_hbm.at[pl.ds(row0, chunk), pl.ds(g * d, d)],
                sw[st])
        for w in wops:
            if w is not None:
                w.wait()

    return k(pa, pb, nbr_t)


def _edge_body(s_ref, et_ref, rt_ref, w12_ref, w3_ref, wr_ref, bias_ref,
               o_ref):
    s2t = s_ref[...].T
    b12 = bias_ref[0:32]
    b3 = bias_ref[32:48]
    br4 = bias_ref[48:64]
    bc = s_ref.shape[0]
    for g in range(4):
        stg = s2t[32 * g:32 * g + 32]
        etg = et_ref[:, g * bc:(g + 1) * bc]
        rtg = rt_ref[:, g * bc:(g + 1) * bc]
        z12 = stg + jnp.dot(w12_ref[...], etg,
                            preferred_element_type=jnp.float32) + b12
        z1 = z12[:16]
        z2 = z12[16:]
        h = (z1 * jax.nn.sigmoid(z1)) * jax.nn.sigmoid(z2)
        t = jnp.dot(w3_ref[...], h, preferred_element_type=jnp.float32) + b3
        gg = jnp.dot(wr_ref[...], rtg, preferred_element_type=jnp.float32) + br4
        o_ref[:, g * bc:(g + 1) * bc] = (t * jax.nn.sigmoid(t)) * gg


def _edge_mlp_t(s128, et, rt, w12, w3, wr, bias_col, blk_lo, n_blk,
                o_prev=None):
    e_total = et.shape[1]
    be = 12800
    bc = be // 4
    small = lambda shp: pl.BlockSpec(shp, lambda b: (0, 0))
    in_specs = [
        pl.BlockSpec((bc, 128), lambda b: (b, 0)),
        pl.BlockSpec((16, be), lambda b: (0, blk_lo + b)),
        pl.BlockSpec((16, be), lambda b: (0, blk_lo + b)),
        small((32, 16)),
        small((16, 16)),
        small((16, 16)),
        small((64, 1)),
    ]
    args = [s128, et, rt, w12, w3, wr, bias_col]
    kwargs = {}
    body = _edge_body
    if o_prev is not None:
        in_specs.append(pl.BlockSpec(memory_space=pl.ANY))
        args.append(o_prev)
        kwargs["input_output_aliases"] = {7: 0}
        body = lambda *refs: _edge_body(*refs[:7], refs[-1])
    return pl.pallas_call(
        body,
        grid=(n_blk,),
        in_specs=in_specs,
        out_specs=pl.BlockSpec((16, be), lambda b: (0, blk_lo + b)),
        out_shape=jax.ShapeDtypeStruct((16, e_total), jnp.float32),
        **kwargs,
    )(*args)


def kernel(atom_fea, edge_ij, nbr_atoms, bonds_r, W1, b1, W2, b2, Wr, br, W3, b3):
    f = atom_fea.shape[1]
    w12 = jnp.concatenate([W1[:, 2 * f:], W2[:, 2 * f:]], axis=0)
    bias_col = jnp.concatenate([b1, b2, b3, br])[:, None]
    pa, pb = _node_proj(atom_fea, W1, W2)
    nbr_t = nbr_atoms.T
    et = edge_ij.T
    rt = bonds_r.T
    chunks = [(0, 32, 0, 8), (32, 64, 8, 8), (64, 88, 16, 6), (88, 100, 22, 3)]
    ss = [_gather_sum_sc(pa, pb, nbr_t, lo, hi) for lo, hi, _, _ in chunks]
    ot = None
    for s128, (_, _, blk_lo, n_blk) in zip(ss, chunks):
        ot = _edge_mlp_t(s128, et, rt, w12, W3, Wr, bias_col,
                         blk_lo, n_blk, o_prev=ot)
    return ot.T
